# Initial kernel scaffold; baseline (speedup 1.0000x reference)
import jax
import jax.numpy as jnp
from jax.experimental import pallas as pl


def kernel(heatmapsx, heatmapsy, labels):
    raise NotImplementedError("write your pallas kernel here")


# speedup vs baseline: 6.4854x; 6.4854x over previous
import jax
import jax.numpy as jnp
from jax import lax
from jax.experimental import pallas as pl
from jax.experimental.pallas import tpu as pltpu
from jax.experimental.pallas import tpu_sc as plsc

_FPS = 25.0
_LOSS_SCALE = 1e-07
_B, _N, _S = 16, 2048, 512
_CN = _N + 16
_L = 16


def _argmax_body(hx_ref, hy_ref, out_ref):
    iot = lax.broadcasted_iota(jnp.int32, (_N, _S), 1)
    x = hx_ref[0]
    mx = jnp.max(x, axis=-1, keepdims=True)
    ax = jnp.min(jnp.where(x == mx, iot, _S), axis=-1)
    y = hy_ref[0]
    my = jnp.max(y, axis=-1, keepdims=True)
    ay = jnp.min(jnp.where(y == my, iot, _S), axis=-1)
    out_ref[0, 0, :] = ax.astype(jnp.float32)
    out_ref[0, 1, :] = ay.astype(jnp.float32)


_argmax_call = pl.pallas_call(
    _argmax_body,
    grid=(_B,),
    in_specs=[
        pl.BlockSpec((1, _N, _S), lambda i: (i, 0, 0)),
        pl.BlockSpec((1, _N, _S), lambda i: (i, 0, 0)),
    ],
    out_specs=pl.BlockSpec((1, 2, _N), lambda i: (i, 0, 0)),
    out_shape=jax.ShapeDtypeStruct((_B, 2, _N), jnp.float32),
)


_GATHER_DNUMS = lax.GatherDimensionNumbers(
    offset_dims=(), collapsed_slice_dims=(0,), start_index_map=(0,))


def _gather16(x, idx):
    return lax.gather(x, idx[:, None], _GATHER_DNUMS, (1,),
                      mode=lax.GatherScatterMode.PROMISE_IN_BOUNDS)


def _sc_loss_body(px_hbm, py_hbm, lx_hbm, ly_hbm, out_hbm,
                  pxv, pyv, lxv, lyv, mfv, cpv, cpxv, cpyv, clxv, clyv, resv):
    c = lax.axis_index("c")
    s = lax.axis_index("s")
    wid = s * 2 + c

    @pl.when(wid < _B)
    def _():
        b = wid
        pltpu.sync_copy(px_hbm.at[b], pxv)
        pltpu.sync_copy(py_hbm.at[b], pyv)
        pltpu.sync_copy(lx_hbm.at[b], lxv)
        pltpu.sync_copy(ly_hbm.at[b], lyv)
        iota = lax.iota(jnp.int32, _L)
        zf = jnp.zeros((_L,), jnp.float32)
        zi = jnp.zeros((_L,), jnp.int32)

        def pass1(i, carry):
            kofs, kvec, den1, den2 = carry
            off = i * _L
            lx = lxv[pl.ds(off, _L)]
            ly = lyv[pl.ds(off, _L)]
            m = jnp.logical_and(lx != 0.0, ly != 0.0)
            mi = jnp.where(m, 1, 0)
            mf = jnp.where(m, 1.0, 0.0)
            incl = mi
            for d in (1, 2, 4, 8):
                sh = _gather16(incl, jnp.maximum(iota - d, 0))
                incl = incl + jnp.where(iota >= d, sh, 0)
            src = zi
            for w in (8, 4, 2, 1):
                probe = _gather16(incl, src + (w - 1))
                src = src + jnp.where(probe < iota + 1, w, 0)
            cpv[pl.ds(kofs, _L)] = off + src
            cpxv[pl.ds(kofs, _L)] = _gather16(pxv[pl.ds(off, _L)], src)
            cpyv[pl.ds(kofs, _L)] = _gather16(pyv[pl.ds(off, _L)], src)
            clxv[pl.ds(kofs, _L)] = _gather16(lx, src)
            clyv[pl.ds(kofs, _L)] = _gather16(ly, src)
            mfv[pl.ds(off, _L)] = mf
            pos = off + iota
            den1 = den1 + jnp.where(pos >= 1, mf, 0.0)
            den2 = den2 + jnp.where(pos >= 2, mf, 0.0)
            cnt = jnp.squeeze(lax.slice(incl, (_L - 1,), (_L,)))
            kvec = kvec + _gather16(incl, jnp.full((_L,), _L - 1, jnp.int32))
            return kofs + cnt, kvec, den1, den2

        kofs, k, den1, den2 = lax.fori_loop(
            0, _N // _L, pass1, (jnp.int32(0), zi, zf, zf))

        def pass2(i, carry):
            vacc, aacc = carry
            j0 = i * _L
            jv = j0 + iota
            p0 = cpv[pl.ds(j0, _L)]
            p1 = cpv[pl.ds(j0 + 1, _L)]
            p2 = cpv[pl.ds(j0 + 2, _L)]
            px0 = cpxv[pl.ds(j0, _L)]
            px1 = cpxv[pl.ds(j0 + 1, _L)]
            px2 = cpxv[pl.ds(j0 + 2, _L)]
            py0 = cpyv[pl.ds(j0, _L)]
            py1 = cpyv[pl.ds(j0 + 1, _L)]
            py2 = cpyv[pl.ds(j0 + 2, _L)]
            qx0 = clxv[pl.ds(j0, _L)]
            qx1 = clxv[pl.ds(j0 + 1, _L)]
            qx2 = clxv[pl.ds(j0 + 2, _L)]
            qy0 = clyv[pl.ds(j0, _L)]
            qy1 = clyv[pl.ds(j0 + 1, _L)]
            qy2 = clyv[pl.ds(j0 + 2, _L)]
            w1 = mfv[pl.ds(j0 + 1, _L)]
            w2 = mfv[pl.ds(j0 + 2, _L)]
            g1 = _FPS / (p1 - p0).astype(jnp.float32)
            g2 = _FPS / (p2 - p1).astype(jnp.float32)
            pvx0 = (px1 - px0) * g1
            pvx1 = (px2 - px1) * g2
            pvy0 = (py1 - py0) * g1
            pvy1 = (py2 - py1) * g2
            avx0 = (qx1 - qx0) * g1
            avx1 = (qx2 - qx1) * g2
            avy0 = (qy1 - qy0) * g1
            avy1 = (qy2 - qy1) * g2
            dvx = pvx0 - avx0
            dvy = pvy0 - avy0
            vsq = dvx * dvx + dvy * dvy
            vacc = vacc + jnp.where(jv < k - 1, vsq * w1, 0.0)
            dax = (pvx1 - pvx0) * g2 - (avx1 - avx0) * g2
            day = (pvy1 - pvy0) * g2 - (avy1 - avy0) * g2
            asq = dax * dax + day * day
            aacc = aacc + jnp.where(jv < k - 2, asq * w2, 0.0)
            return vacc, aacc

        vacc, aacc = lax.fori_loop(0, _N // _L, pass2, (zf, zf))
        resv[0, :] = vacc
        resv[1, :] = aacc
        resv[2, :] = den1
        resv[3, :] = den2
        pltpu.sync_copy(resv, out_hbm.at[b])


_sc_loss_call = pl.kernel(
    _sc_loss_body,
    out_type=jax.ShapeDtypeStruct((_B, 4, _L), jnp.float32),
    mesh=plsc.VectorSubcoreMesh(core_axis_name="c", subcore_axis_name="s"),
    scratch_types=[
        pltpu.VMEM((_N,), jnp.float32),
        pltpu.VMEM((_N,), jnp.float32),
        pltpu.VMEM((_N,), jnp.float32),
        pltpu.VMEM((_N,), jnp.float32),
        pltpu.VMEM((_CN,), jnp.float32),
        pltpu.VMEM((_CN,), jnp.int32),
        pltpu.VMEM((_CN,), jnp.float32),
        pltpu.VMEM((_CN,), jnp.float32),
        pltpu.VMEM((_CN,), jnp.float32),
        pltpu.VMEM((_CN,), jnp.float32),
        pltpu.VMEM((4, _L), jnp.float32),
    ],
)


def kernel(heatmapsx, heatmapsy, labels):
    coords = _argmax_call(heatmapsx, heatmapsy)
    px = coords[:, 0, :]
    py = coords[:, 1, :]
    lx = labels[:, :, 0]
    ly = labels[:, :, 1]
    partials = _sc_loss_call(px, py, lx, ly)
    sums = jnp.sum(partials, axis=(0, 2))
    total = sums[0] / sums[2] + 0.1 * sums[1] / sums[3]
    return _LOSS_SCALE * total
